# baseline (speedup 1.0000x reference)
import jax
import jax.numpy as jnp
from jax.experimental import pallas as pl

SDIM, VDIM, RBF_N, CUTOFF, L = 64, 16, 16, 5.0, 5


def _edge_attrs(pos, ei):
    src, dst = ei[0], ei[1]
    r = pos[dst] - pos[src]
    d = jnp.sqrt(jnp.clip(jnp.sum(r * r, axis=-1), 1e-6, None))
    rn = r / d[:, None]
    return src, dst, d, rn


def _rbf(d, use_env):
    offs = jnp.linspace(0.0, CUTOFF, RBF_N)
    gamma = 10.0 / (CUTOFF ** 2)
    feat = jnp.exp(-gamma * (d[:, None] - offs[None, :]) ** 2)
    if use_env:
        env = 0.5 * (jnp.cos(jnp.pi * jnp.clip(d / CUTOFF, 0.0, 1.0)) + 1.0)
        feat = feat * env[:, None]
    return feat


def _ln(x):
    m = jnp.mean(x, axis=-1, keepdims=True)
    v = jnp.var(x, axis=-1, keepdims=True)
    return (x - m) / jnp.sqrt(v + 1e-5)


def _mp(s, v, src, dst, rbf, rn, Wf, Wu, bu, Wv):
    n = s.shape[0]
    filt = rbf @ Wf
    fs = filt[:, :SDIM]
    fv1 = filt[:, SDIM:SDIM + VDIM]
    fv2 = filt[:, SDIM + VDIM:]
    ms = s[src] * fs
    mv = v[src] * fv1[:, None, :] + rn[:, :, None] * fv2[:, None, :]
    agg_s = jax.ops.segment_sum(ms, dst, num_segments=n)
    agg_v = jax.ops.segment_sum(mv, dst, num_segments=n)
    deg = jax.ops.segment_sum(jnp.ones((src.shape[0],), dtype=s.dtype), dst, num_segments=n)
    agg_v = agg_v / jnp.maximum(deg, 1.0)[:, None, None]
    vn = jnp.sqrt(jnp.sum(agg_v * agg_v, axis=1) + 1e-6)
    ds = jnp.tanh(jnp.concatenate([agg_s, vn], axis=-1) @ Wu + bu)
    s = _ln(s + ds)
    v = v + agg_v @ Wv
    return s, v


def _proj_body(s_ref, vf_ref, wc_ref, wsc_ref, sa_ref, scv_ref):
    scv_ref[...] = vf_ref[...] @ wc_ref[...]
    sa_ref[...] = s_ref[...] @ wsc_ref[...]


def _final_proj(s, v, Wc, Wsc):
    n = s.shape[0]
    vf = v.reshape(n * 3, VDIM)
    blk = 2000
    grid = n // blk
    sa, scv = pl.pallas_call(
        _proj_body,
        grid=(grid,),
        in_specs=[
            pl.BlockSpec((blk, SDIM), lambda i: (i, 0)),
            pl.BlockSpec((3 * blk, VDIM), lambda i: (i, 0)),
            pl.BlockSpec((VDIM, 1), lambda i: (0, 0)),
            pl.BlockSpec((SDIM, 16), lambda i: (0, 0)),
        ],
        out_specs=[
            pl.BlockSpec((blk, 16), lambda i: (i, 0)),
            pl.BlockSpec((3 * blk, 1), lambda i: (i, 0)),
        ],
        out_shape=[
            jax.ShapeDtypeStruct((n, 16), jnp.float32),
            jax.ShapeDtypeStruct((n * 3, 1), jnp.float32),
        ],
    )(s, vf, Wc, Wsc)
    return scv.reshape(n, 3), sa


def kernel(x, t, pos, edge_index_local, edge_index_global, batch, Wt, bt, Wa, ba, Wat, bat, Wf_l, Wu_l, bu_l, Wv_l, Wf_g, Wu_g, bu_g, Wv_g, Wc, Wsc):
    src_l, dst_l, d_l, rn_l = _edge_attrs(pos, edge_index_local)
    src_g, dst_g, d_g, rn_g = _edge_attrs(pos, edge_index_global)
    rbf_l = _rbf(d_l, True)
    rbf_g = _rbf(d_g, False)
    te = (t @ Wt + bt)[batch]
    s = x @ Wa + ba
    s = (s + te) @ Wat + bat
    v = jnp.zeros((x.shape[0], 3, VDIM), dtype=s.dtype)
    for l in range(L):
        s, v = _mp(s, v, src_l, dst_l, rbf_l, rn_l, Wf_l[l], Wu_l[l], bu_l[l], Wv_l[l])
        s, v = _mp(s, v, src_g, dst_g, rbf_g, rn_g, Wf_g[l], Wu_g[l], bu_g[l], Wv_g[l])
    return _final_proj(s, v, Wc, Wsc)


# speedup vs baseline: 19.4859x; 19.4859x over previous
import functools

import jax
import jax.numpy as jnp
from jax import lax
from jax.experimental import pallas as pl
from jax.experimental.pallas import tpu as pltpu
from jax.experimental.pallas import tpu_sc as plsc

N = 50000
E = 800000
B = 256
SDIM, VDIM = 64, 16
RBF_N = 16
CUTOFF = 5.0
L = 5
NT = 16

NC, NS = 2, 16
C = 128
NCHUNK = E // C
HALF_NCHUNK = NCHUNK // 2
ZC = 128
NFULL = N // ZC
NREM = N - NFULL * ZC
REM_TILE = NFULL % NS

_SC_MESH = dict(core_axis_name="c", subcore_axis_name="s",
                num_cores=NC, num_subcores=NS)


def _strided_count(rel, nchunk, stride):
    return lax.max(0, (nchunk - rel + stride - 1) // stride)


def _acc_zero(s, zb, accs):
    cnt = _strided_count(s, NFULL, NS)

    def body(i, _):
        base = (s + i * NS) * ZC
        for acc in accs:
            pltpu.sync_copy(zb, acc.at[pl.ds(base, ZC)])
        return 0

    lax.fori_loop(0, cnt, body, 0)

    @pl.when(s == REM_TILE)
    def _():
        for acc in accs:
            pltpu.sync_copy(zb.at[pl.ds(0, NREM)],
                            acc.at[pl.ds(NFULL * ZC, NREM)])


def _acc_writeback(s, zb, pairs):
    cnt = _strided_count(s, NFULL, NS)

    def body(i, _):
        base = (s + i * NS) * ZC
        for acc, out in pairs:
            pltpu.sync_copy(acc.at[pl.ds(base, ZC)], zb)
            pltpu.sync_copy(zb, out.at[pl.ds(base, ZC)])
        return 0

    lax.fori_loop(0, cnt, body, 0)

    @pl.when(s == REM_TILE)
    def _():
        for acc, out in pairs:
            pltpu.sync_copy(acc.at[pl.ds(NFULL * ZC, NREM)],
                            zb.at[pl.ds(0, NREM)])
            pltpu.sync_copy(zb.at[pl.ds(0, NREM)],
                            out.at[pl.ds(NFULL * ZC, NREM)])


def _posdeg_body(ei, pos16, psrc, pdst, dega, degb,
                 sidx, didx, prs, prd, ones_v, zb, acc, sem):
    c = lax.axis_index("c")
    s = lax.axis_index("s")
    wid = c * NS + s

    zv = jnp.zeros((16,), jnp.float32)
    ov = jnp.ones((16,), jnp.float32)
    for r in range(ZC):
        zb[r] = zv
    for r in range(C):
        ones_v[r] = ov
    _acc_zero(s, zb, [acc])
    plsc.subcore_barrier()

    cnt = _strided_count(wid, NCHUNK, NC * NS)

    def chunk(i, _):
        g = wid + i * (NC * NS)
        base = g * C
        pltpu.sync_copy(ei.at[0, pl.ds(base, C)], sidx)
        pltpu.sync_copy(ei.at[1, pl.ds(base, C)], didx)
        pltpu.async_copy(pos16.at[sidx], prs, sem).wait()
        pltpu.async_copy(pos16.at[didx], prd, sem).wait()
        pltpu.sync_copy(prs, psrc.at[pl.ds(base, C)])
        pltpu.sync_copy(prd, pdst.at[pl.ds(base, C)])
        pltpu.sync_copy(ones_v, acc.at[didx], add=True)
        return 0

    lax.fori_loop(0, cnt, chunk, 0)
    plsc.subcore_barrier()

    for cc, out in ((0, dega), (1, degb)):
        @pl.when(c == cc)
        def _(out=out):
            _acc_writeback(s, zb, [(acc, out)])


_posdeg = pl.kernel(
    _posdeg_body,
    out_type=[jax.ShapeDtypeStruct((E, 16), jnp.float32),
              jax.ShapeDtypeStruct((E, 16), jnp.float32),
              jax.ShapeDtypeStruct((N, 16), jnp.float32),
              jax.ShapeDtypeStruct((N, 16), jnp.float32)],
    mesh=plsc.VectorSubcoreMesh(**_SC_MESH),
    compiler_params=pltpu.CompilerParams(use_tc_tiling_on_sc=False),
    scratch_types=[pltpu.VMEM((C,), jnp.int32),
                   pltpu.VMEM((C,), jnp.int32),
                   pltpu.VMEM((C, 16), jnp.float32),
                   pltpu.VMEM((C, 16), jnp.float32),
                   pltpu.VMEM((C, 16), jnp.float32),
                   pltpu.VMEM((ZC, 16), jnp.float32),
                   pltpu.VMEM_SHARED((N, 16), jnp.float32),
                   pltpu.SemaphoreType.DMA],
)


def _spass_body(ei, s0, s1, fs0, fs1, agg0, agg1,
                sidx, didx, rows, fsb, zb, acc, sem):
    c = lax.axis_index("c")
    s = lax.axis_index("s")

    zv = jnp.zeros((16,), jnp.float32)
    for r in range(ZC):
        zb[r, pl.ds(0, 16)] = zv
        zb[r, pl.ds(16, 16)] = zv
    _acc_zero(s, zb, [acc])
    plsc.subcore_barrier()

    cnt = _strided_count(s, NCHUNK, NS)

    def make_chunk(tab, filt):
        def chunk(i, _):
            g = s + i * NS
            base = g * C
            pltpu.sync_copy(ei.at[0, pl.ds(base, C)], sidx)
            pltpu.sync_copy(ei.at[1, pl.ds(base, C)], didx)
            pltpu.async_copy(tab.at[sidx], rows, sem).wait()
            pltpu.sync_copy(filt.at[pl.ds(base, C)], fsb)

            def rbody(r4, _):
                for k in range(4):
                    r = r4 * 4 + k
                    rows[r, pl.ds(0, 16)] = rows[r, pl.ds(0, 16)] * fsb[r, pl.ds(0, 16)]
                    rows[r, pl.ds(16, 16)] = rows[r, pl.ds(16, 16)] * fsb[r, pl.ds(16, 16)]
                return 0

            lax.fori_loop(0, C // 4, rbody, 0)
            pltpu.sync_copy(rows, acc.at[didx], add=True)
            return 0
        return chunk

    for cc, tab, filt in ((0, s0, fs0), (1, s1, fs1)):
        @pl.when(c == cc)
        def _(tab=tab, filt=filt):
            lax.fori_loop(0, cnt, make_chunk(tab, filt), 0)

    plsc.subcore_barrier()
    for cc, out in ((0, agg0), (1, agg1)):
        @pl.when(c == cc)
        def _(out=out):
            _acc_writeback(s, zb, [(acc, out)])


_spass = pl.kernel(
    _spass_body,
    out_type=[jax.ShapeDtypeStruct((N, 32), jnp.float32),
              jax.ShapeDtypeStruct((N, 32), jnp.float32)],
    mesh=plsc.VectorSubcoreMesh(**_SC_MESH),
    compiler_params=pltpu.CompilerParams(use_tc_tiling_on_sc=False),
    scratch_types=[pltpu.VMEM((C,), jnp.int32),
                   pltpu.VMEM((C,), jnp.int32),
                   pltpu.VMEM((C, 32), jnp.float32),
                   pltpu.VMEM((C, 32), jnp.float32),
                   pltpu.VMEM((ZC, 32), jnp.float32),
                   pltpu.VMEM_SHARED((N, 32), jnp.float32),
                   pltpu.SemaphoreType.DMA],
)


def _vpass_body(ei, v0, v1, v2, fv1, fv2, rr0, rr1, rr2,
                a0, a1, a2a, a2b,
                sidx, didx, vr, f1b, f2b, rrb, zb, accA, accB, sem):
    c = lax.axis_index("c")
    s = lax.axis_index("s")

    zv = jnp.zeros((16,), jnp.float32)
    for r in range(ZC):
        zb[r] = zv
    _acc_zero(s, zb, [accA, accB])
    plsc.subcore_barrier()

    def make_chunk(g0, tab, rr, acc):
        def chunk(i, _):
            g = g0 + s + i * NS
            base = g * C
            pltpu.sync_copy(ei.at[0, pl.ds(base, C)], sidx)
            pltpu.sync_copy(ei.at[1, pl.ds(base, C)], didx)
            pltpu.async_copy(tab.at[sidx], vr, sem).wait()
            pltpu.sync_copy(fv1.at[pl.ds(base, C)], f1b)
            pltpu.sync_copy(fv2.at[pl.ds(base, C)], f2b)
            pltpu.sync_copy(rr.at[pl.ds(base, C)], rrb)

            def rbody(r4, _):
                for k in range(4):
                    r = r4 * 4 + k
                    vr[r] = vr[r] * f1b[r] + rrb[r] * f2b[r]
                return 0

            lax.fori_loop(0, C // 4, rbody, 0)
            pltpu.sync_copy(vr, acc.at[didx], add=True)
            return 0
        return chunk

    cnt_full = _strided_count(s, NCHUNK, NS)
    cnt_half = _strided_count(s, HALF_NCHUNK, NS)

    for cc, tab, rr in ((0, v0, rr0), (1, v1, rr1)):
        @pl.when(c == cc)
        def _(tab=tab, rr=rr):
            lax.fori_loop(0, cnt_full, make_chunk(0, tab, rr, accA), 0)
    for cc in (0, 1):
        @pl.when(c == cc)
        def _(cc=cc):
            lax.fori_loop(0, cnt_half,
                          make_chunk(cc * HALF_NCHUNK, v2, rr2, accB), 0)

    plsc.subcore_barrier()
    for cc, outA, outB in ((0, a0, a2a), (1, a1, a2b)):
        @pl.when(c == cc)
        def _(outA=outA, outB=outB):
            _acc_writeback(s, zb, [(accA, outA), (accB, outB)])


_vpass = pl.kernel(
    _vpass_body,
    out_type=[jax.ShapeDtypeStruct((N, 16), jnp.float32)] * 4,
    mesh=plsc.VectorSubcoreMesh(**_SC_MESH),
    compiler_params=pltpu.CompilerParams(use_tc_tiling_on_sc=False),
    scratch_types=[pltpu.VMEM((C,), jnp.int32),
                   pltpu.VMEM((C,), jnp.int32),
                   pltpu.VMEM((C, 16), jnp.float32),
                   pltpu.VMEM((C, 16), jnp.float32),
                   pltpu.VMEM((C, 16), jnp.float32),
                   pltpu.VMEM((C, 16), jnp.float32),
                   pltpu.VMEM((ZC, 16), jnp.float32),
                   pltpu.VMEM_SHARED((N, 16), jnp.float32),
                   pltpu.VMEM_SHARED((N, 16), jnp.float32),
                   pltpu.SemaphoreType.DMA],
)


RN_BLK = 2000
RE_BLK = 2000


def _init_body(x, t, batch, Wt, bt, Wa, ba, Wat, bat, s0, s1):
    te_tab = jnp.dot(t[...], Wt[...], preferred_element_type=jnp.float32) + bt[...]
    onehot = (batch[...] == lax.broadcasted_iota(jnp.int32, (RN_BLK, B), 1)
              ).astype(jnp.float32)
    te = jnp.dot(onehot, te_tab, preferred_element_type=jnp.float32)
    sa = jnp.dot(x[...], Wa[...], preferred_element_type=jnp.float32) + ba[...]
    sfull = jnp.dot(sa + te, Wat[...], preferred_element_type=jnp.float32) + bat[...]
    s0[...] = sfull[:, :32]
    s1[...] = sfull[:, 32:]


def _init_s(x, t, batch, Wt, bt, Wa, ba, Wat, bat):
    grid = N // RN_BLK
    return pl.pallas_call(
        _init_body,
        grid=(grid,),
        in_specs=[
            pl.BlockSpec((RN_BLK, NT), lambda i: (i, 0)),
            pl.BlockSpec((B, 1), lambda i: (0, 0)),
            pl.BlockSpec((RN_BLK, 1), lambda i: (i, 0)),
            pl.BlockSpec((1, SDIM), lambda i: (0, 0)),
            pl.BlockSpec((1, SDIM), lambda i: (0, 0)),
            pl.BlockSpec((NT, SDIM), lambda i: (0, 0)),
            pl.BlockSpec((1, SDIM), lambda i: (0, 0)),
            pl.BlockSpec((SDIM, SDIM), lambda i: (0, 0)),
            pl.BlockSpec((1, SDIM), lambda i: (0, 0)),
        ],
        out_specs=[pl.BlockSpec((RN_BLK, 32), lambda i: (i, 0)),
                   pl.BlockSpec((RN_BLK, 32), lambda i: (i, 0))],
        out_shape=[jax.ShapeDtypeStruct((N, 32), jnp.float32)] * 2,
    )(x, t, batch.reshape(N, 1), Wt, bt.reshape(1, SDIM), Wa,
      ba.reshape(1, SDIM), Wat, bat.reshape(1, SDIM))


def _geom_body(use_env, ps, pd, rbf, rr0, rr1, rr2):
    r = pd[...] - ps[...]
    d2 = jnp.sum(r * r, axis=1, keepdims=True)
    d = jnp.sqrt(jnp.clip(d2, 1e-6, None))
    rn = r / d
    offs = (lax.broadcasted_iota(jnp.int32, (1, RBF_N), 1).astype(jnp.float32)
            * (CUTOFF / (RBF_N - 1)))
    gamma = 10.0 / (CUTOFF ** 2)
    feat = jnp.exp(-gamma * (d - offs) ** 2)
    if use_env:
        env = 0.5 * (jnp.cos(jnp.pi * jnp.clip(d / CUTOFF, 0.0, 1.0)) + 1.0)
        feat = feat * env
    rbf[...] = feat
    ones16 = jnp.ones((1, 16), jnp.float32)
    rr0[...] = rn[:, 0:1] * ones16
    rr1[...] = rn[:, 1:2] * ones16
    rr2[...] = rn[:, 2:3] * ones16


def _geom(ps, pd, use_env):
    grid = E // RE_BLK
    return pl.pallas_call(
        functools.partial(_geom_body, use_env),
        grid=(grid,),
        in_specs=[pl.BlockSpec((RE_BLK, 16), lambda i: (i, 0))] * 2,
        out_specs=[pl.BlockSpec((RE_BLK, 16), lambda i: (i, 0))] * 4,
        out_shape=[jax.ShapeDtypeStruct((E, 16), jnp.float32)] * 4,
    )(ps, pd)


def _deginv_body(da, db, out):
    out[...] = 1.0 / jnp.maximum(da[...] + db[...], 1.0)


def _deginv(da, db):
    grid = N // RN_BLK
    return pl.pallas_call(
        _deginv_body,
        grid=(grid,),
        in_specs=[pl.BlockSpec((RN_BLK, 16), lambda i: (i, 0))] * 2,
        out_specs=pl.BlockSpec((RN_BLK, 16), lambda i: (i, 0)),
        out_shape=jax.ShapeDtypeStruct((N, 16), jnp.float32),
    )(da, db)


def _filt_body(rbf, Wf, fs0, fs1, fv1, fv2):
    filt = jnp.dot(rbf[...], Wf[...], preferred_element_type=jnp.float32)
    fs0[...] = filt[:, :32]
    fs1[...] = filt[:, 32:64]
    fv1[...] = filt[:, 64:80]
    fv2[...] = filt[:, 80:96]


def _filt(rbf, Wf):
    grid = E // RE_BLK
    return pl.pallas_call(
        _filt_body,
        grid=(grid,),
        in_specs=[pl.BlockSpec((RE_BLK, RBF_N), lambda i: (i, 0)),
                  pl.BlockSpec((RBF_N, SDIM + 2 * VDIM), lambda i: (0, 0))],
        out_specs=[pl.BlockSpec((RE_BLK, 32), lambda i: (i, 0)),
                   pl.BlockSpec((RE_BLK, 32), lambda i: (i, 0)),
                   pl.BlockSpec((RE_BLK, 16), lambda i: (i, 0)),
                   pl.BlockSpec((RE_BLK, 16), lambda i: (i, 0))],
        out_shape=[jax.ShapeDtypeStruct((E, 32), jnp.float32),
                   jax.ShapeDtypeStruct((E, 32), jnp.float32),
                   jax.ShapeDtypeStruct((E, 16), jnp.float32),
                   jax.ShapeDtypeStruct((E, 16), jnp.float32)],
    )(rbf, Wf)


def _node_body(s0, s1, v0, v1, v2, g0, g1, a0, a1, a2a, a2b, dinv,
               Wu, bu, Wv, s0o, s1o, v0o, v1o, v2o):
    di = dinv[...]
    b0 = a0[...] * di
    b1 = a1[...] * di
    b2 = (a2a[...] + a2b[...]) * di
    vn = jnp.sqrt(b0 * b0 + b1 * b1 + b2 * b2 + 1e-6)
    wu = Wu[...]
    ds = jnp.tanh(
        jnp.dot(g0[...], wu[0:32, :], preferred_element_type=jnp.float32)
        + jnp.dot(g1[...], wu[32:64, :], preferred_element_type=jnp.float32)
        + jnp.dot(vn, wu[64:80, :], preferred_element_type=jnp.float32)
        + bu[...])
    h0 = s0[...] + ds[:, :32]
    h1 = s1[...] + ds[:, 32:]
    m = (jnp.sum(h0, axis=1, keepdims=True)
         + jnp.sum(h1, axis=1, keepdims=True)) * (1.0 / SDIM)
    d0 = h0 - m
    d1 = h1 - m
    var = (jnp.sum(d0 * d0, axis=1, keepdims=True)
           + jnp.sum(d1 * d1, axis=1, keepdims=True)) * (1.0 / SDIM)
    inv = 1.0 / jnp.sqrt(var + 1e-5)
    s0o[...] = d0 * inv
    s1o[...] = d1 * inv
    wv = Wv[...]
    v0o[...] = v0[...] + jnp.dot(b0, wv, preferred_element_type=jnp.float32)
    v1o[...] = v1[...] + jnp.dot(b1, wv, preferred_element_type=jnp.float32)
    v2o[...] = v2[...] + jnp.dot(b2, wv, preferred_element_type=jnp.float32)


def _node(s0, s1, v0, v1, v2, g0, g1, a0, a1, a2a, a2b, dinv, Wu, bu, Wv):
    grid = N // RN_BLK
    n32 = pl.BlockSpec((RN_BLK, 32), lambda i: (i, 0))
    n16 = pl.BlockSpec((RN_BLK, 16), lambda i: (i, 0))
    return pl.pallas_call(
        _node_body,
        grid=(grid,),
        in_specs=[n32, n32, n16, n16, n16, n32, n32, n16, n16, n16, n16, n16,
                  pl.BlockSpec((SDIM + VDIM, SDIM), lambda i: (0, 0)),
                  pl.BlockSpec((1, SDIM), lambda i: (0, 0)),
                  pl.BlockSpec((VDIM, VDIM), lambda i: (0, 0))],
        out_specs=[n32, n32, n16, n16, n16],
        out_shape=[jax.ShapeDtypeStruct((N, 32), jnp.float32),
                   jax.ShapeDtypeStruct((N, 32), jnp.float32),
                   jax.ShapeDtypeStruct((N, 16), jnp.float32),
                   jax.ShapeDtypeStruct((N, 16), jnp.float32),
                   jax.ShapeDtypeStruct((N, 16), jnp.float32)],
    )(s0, s1, v0, v1, v2, g0, g1, a0, a1, a2a, a2b, dinv,
      Wu, bu.reshape(1, SDIM), Wv)


def _final_body(s0, s1, v0, v1, v2, W30, W31, W32, Wsc, coords, atoms):
    coords[...] = (
        jnp.dot(v0[...], W30[...], preferred_element_type=jnp.float32)
        + jnp.dot(v1[...], W31[...], preferred_element_type=jnp.float32)
        + jnp.dot(v2[...], W32[...], preferred_element_type=jnp.float32))
    wsc = Wsc[...]
    atoms[...] = (
        jnp.dot(s0[...], wsc[0:32, :], preferred_element_type=jnp.float32)
        + jnp.dot(s1[...], wsc[32:64, :], preferred_element_type=jnp.float32))


def _final(s0, s1, v0, v1, v2, Wc, Wsc):
    eye = jnp.eye(16, dtype=jnp.float32)
    W3 = [Wc.reshape(16, 1) * eye[c:c + 1, :] for c in range(3)]
    grid = N // RN_BLK
    n32 = pl.BlockSpec((RN_BLK, 32), lambda i: (i, 0))
    n16 = pl.BlockSpec((RN_BLK, 16), lambda i: (i, 0))
    w16 = pl.BlockSpec((16, 16), lambda i: (0, 0))
    return pl.pallas_call(
        _final_body,
        grid=(grid,),
        in_specs=[n32, n32, n16, n16, n16, w16, w16, w16,
                  pl.BlockSpec((SDIM, NT), lambda i: (0, 0))],
        out_specs=[n16, n16],
        out_shape=[jax.ShapeDtypeStruct((N, 16), jnp.float32),
                   jax.ShapeDtypeStruct((N, 16), jnp.float32)],
    )(s0, s1, v0, v1, v2, W3[0], W3[1], W3[2], Wsc)


def kernel(x, t, pos, edge_index_local, edge_index_global, batch,
           Wt, bt, Wa, ba, Wat, bat, Wf_l, Wu_l, bu_l, Wv_l,
           Wf_g, Wu_g, bu_g, Wv_g, Wc, Wsc):
    pos16 = jnp.pad(pos, ((0, 0), (0, 13)))

    s0, s1 = _init_s(x, t, batch, Wt, bt, Wa, ba, Wat, bat)

    edges = {}
    for name, ei, use_env in (("l", edge_index_local, True),
                              ("g", edge_index_global, False)):
        psrc, pdst, dega, degb = _posdeg(ei, pos16)
        rbf, rr0, rr1, rr2 = _geom(psrc, pdst, use_env)
        dinv = _deginv(dega, degb)
        edges[name] = (ei, rbf, rr0, rr1, rr2, dinv)

    v0 = jnp.zeros((N, 16), jnp.float32)
    v1 = jnp.zeros((N, 16), jnp.float32)
    v2 = jnp.zeros((N, 16), jnp.float32)

    params = {"l": (Wf_l, Wu_l, bu_l, Wv_l), "g": (Wf_g, Wu_g, bu_g, Wv_g)}
    for l in range(L):
        for name in ("l", "g"):
            ei, rbf, rr0, rr1, rr2, dinv = edges[name]
            Wf, Wu, bu, Wv = params[name]
            fs0, fs1, fv1, fv2 = _filt(rbf, Wf[l])
            g0, g1 = _spass(ei, s0, s1, fs0, fs1)
            a0, a1, a2a, a2b = _vpass(ei, v0, v1, v2, fv1, fv2, rr0, rr1, rr2)
            s0, s1, v0, v1, v2 = _node(s0, s1, v0, v1, v2, g0, g1,
                                       a0, a1, a2a, a2b, dinv,
                                       Wu[l], bu[l], Wv[l])

    coords16, atoms = _final(s0, s1, v0, v1, v2, Wc, Wsc)
    return coords16[:, :3], atoms
